# Initial kernel scaffold; baseline (speedup 1.0000x reference)
import jax
import jax.numpy as jnp
from jax.experimental import pallas as pl


def kernel(x, edge_index, W1, b1, W2, b2):
    raise NotImplementedError("write your pallas kernel here")


# speedup vs baseline: 13.4495x; 13.4495x over previous
import functools

import jax
import jax.numpy as jnp
from jax import lax
from jax.experimental import pallas as pl
from jax.experimental.pallas import tpu as pltpu
from jax.experimental.pallas import tpu_sc as plsc

N = 10000
D = 128
E = 320000

NP = 10240
NW = 32
CHUNK = 128
EPW = (E + NW - 1) // NW
NCHUNK = (EPW + CHUNK - 1) // CHUNK
EPAD = NW * NCHUNK * CHUNK
RPT = NP // 16


def _deg_body(dstp_hbm, ones_hbm, zros_hbm, out_hbm, idx_v, ones_v, acc_sh):
    c = lax.axis_index("c")
    s = lax.axis_index("s")
    wid = c * 16 + s
    pltpu.sync_copy(dstp_hbm.at[wid], idx_v)
    pltpu.sync_copy(ones_hbm, ones_v)
    pltpu.sync_copy(zros_hbm.at[pl.ds(s * RPT, RPT)], acc_sh.at[pl.ds(s * RPT, RPT)])
    plsc.subcore_barrier()

    def body(j, carry):
        pltpu.sync_copy(ones_v, acc_sh.at[idx_v.at[j]], add=True)
        return carry

    lax.fori_loop(0, NCHUNK, body, 0)
    plsc.subcore_barrier()
    pltpu.sync_copy(
        acc_sh.at[pl.ds(s * RPT, RPT)],
        out_hbm.at[pl.ds(c * NP + s * RPT, RPT)],
    )


def _agg_body(xs_hbm, srcp_hbm, dstp_hbm, zros_hbm, out_hbm,
              src_v, dst_v, rows0, rows1, sem0, sem1, acc_sh):
    c = lax.axis_index("c")
    s = lax.axis_index("s")
    wid = c * 16 + s
    pltpu.sync_copy(srcp_hbm.at[wid], src_v)
    pltpu.sync_copy(dstp_hbm.at[wid], dst_v)
    pltpu.sync_copy(zros_hbm.at[pl.ds(s * RPT, RPT)], acc_sh.at[pl.ds(s * RPT, RPT)])
    plsc.subcore_barrier()

    def body(j, carry):
        pltpu.async_copy(xs_hbm.at[src_v.at[j]], rows0, sem0).wait()
        pltpu.sync_copy(rows0, acc_sh.at[dst_v.at[j]], add=True)
        return carry

    lax.fori_loop(0, NCHUNK, body, 0)
    plsc.subcore_barrier()
    pltpu.sync_copy(
        acc_sh.at[pl.ds(s * RPT, RPT)],
        out_hbm.at[pl.ds(c * NP + s * RPT, RPT)],
    )


def _tc1_body(deg_ref, x_ref, xs_ref):
    deg = deg_ref[0:N, 0:1] + deg_ref[NP:NP + N, 0:1] + 1.0
    dinv = lax.rsqrt(deg)
    xs_ref[0:N, :] = x_ref[...] * dinv
    xs_ref[N:NP, :] = jnp.zeros((NP - N, D), jnp.float32)


def _tc2_body(acc_ref, deg_ref, x_ref, w_ref, b_ref, h_ref, xs_ref):
    deg = deg_ref[0:N, 0:1] + deg_ref[NP:NP + N, 0:1] + 1.0
    dinv = lax.rsqrt(deg)
    aggs = acc_ref[0:N, :] + acc_ref[NP:NP + N, :]
    agg = dinv * aggs + (dinv * dinv) * x_ref[...]
    out = jnp.dot(agg, w_ref[...], preferred_element_type=jnp.float32) + b_ref[...]
    h = jnp.maximum(out, 0.0)
    h_ref[...] = h
    xs_ref[0:N, :] = h * dinv
    xs_ref[N:NP, :] = jnp.zeros((NP - N, D), jnp.float32)


def _tc3_body(acc_ref, deg_ref, h_ref, w_ref, b_ref, out_ref):
    deg = deg_ref[0:N, 0:1] + deg_ref[NP:NP + N, 0:1] + 1.0
    dinv = lax.rsqrt(deg)
    aggs = acc_ref[0:N, :] + acc_ref[NP:NP + N, :]
    agg = dinv * aggs + (dinv * dinv) * h_ref[...]
    o = jnp.dot(agg, w_ref[...], preferred_element_type=jnp.float32) + b_ref[...]
    m = jnp.max(o, axis=-1, keepdims=True)
    u = o - m
    lse = jnp.log(jnp.sum(jnp.exp(u), axis=-1, keepdims=True))
    out_ref[...] = u - lse


_DEG_SCRATCH = [
    pltpu.VMEM((NCHUNK, CHUNK), jnp.int32),
    pltpu.VMEM((CHUNK, D), jnp.float32),
    pltpu.VMEM_SHARED((NP, D), jnp.float32),
]
_AGG_SCRATCH = [
    pltpu.VMEM((NCHUNK, CHUNK), jnp.int32),
    pltpu.VMEM((NCHUNK, CHUNK), jnp.int32),
    pltpu.VMEM((CHUNK, D), jnp.float32),
    pltpu.VMEM((CHUNK, D), jnp.float32),
    pltpu.SemaphoreType.DMA,
    pltpu.SemaphoreType.DMA,
    pltpu.VMEM_SHARED((NP, D), jnp.float32),
]


@functools.cache
def _sc_kernels():
    mesh = plsc.VectorSubcoreMesh(core_axis_name="c", subcore_axis_name="s")
    deg_k = pl.kernel(
        _deg_body,
        out_type=jax.ShapeDtypeStruct((2 * NP, D), jnp.float32),
        mesh=mesh,
        scratch_types=_DEG_SCRATCH,
    )
    agg_k = pl.kernel(
        _agg_body,
        out_type=jax.ShapeDtypeStruct((2 * NP, D), jnp.float32),
        mesh=mesh,
        scratch_types=_AGG_SCRATCH,
    )
    return deg_k, agg_k

_tc1 = pl.pallas_call(
    _tc1_body, out_shape=jax.ShapeDtypeStruct((NP, D), jnp.float32))
_tc2 = pl.pallas_call(
    _tc2_body,
    out_shape=(jax.ShapeDtypeStruct((N, D), jnp.float32),
               jax.ShapeDtypeStruct((NP, D), jnp.float32)))
_tc3 = pl.pallas_call(
    _tc3_body, out_shape=jax.ShapeDtypeStruct((N, D), jnp.float32))


def kernel(x, edge_index, W1, b1, W2, b2):
    src = edge_index[0]
    dst = edge_index[1]
    pad = jnp.full((EPAD - E,), N, dtype=jnp.int32)
    srcp = jnp.concatenate([src, pad]).reshape(NW, NCHUNK, CHUNK)
    dstp = jnp.concatenate([dst, pad]).reshape(NW, NCHUNK, CHUNK)

    onesw = jnp.ones((CHUNK, D), jnp.float32)
    zbig = jnp.zeros((NP, D), jnp.float32)
    b1r = b1.reshape(1, D)
    b2r = b2.reshape(1, D)

    deg_kernel, agg_kernel = _sc_kernels()
    deg = deg_kernel(dstp, onesw, zbig)
    xs1 = _tc1(deg, x)
    acc1 = agg_kernel(xs1, srcp, dstp, zbig)
    h, xs2 = _tc2(acc1, deg, x, W1, b1r)
    acc2 = agg_kernel(xs2, srcp, dstp, zbig)
    return _tc3(acc2, deg, h, W2, b2r)
